# baseline (speedup 1.0000x reference)
import functools

import jax
import jax.numpy as jnp
from jax import lax
from jax.experimental import pallas as pl
from jax.experimental.pallas import tpu as pltpu
from jax.experimental.pallas import tpu_sc as plsc

DIM = 32
NUM_WORKERS = 32
CHUNK = 1600


def _emb_kernel(n_total):
    per_w = n_total // NUM_WORKERS
    n_chunks = per_w // CHUNK
    mesh = plsc.VectorSubcoreMesh(core_axis_name="c", subcore_axis_name="s")

    @functools.partial(
        pl.kernel,
        mesh=mesh,
        out_type=jax.ShapeDtypeStruct((n_total, DIM), jnp.float32),
        scratch_types=[
            pltpu.VMEM((n_chunks, CHUNK), jnp.int32),
            pltpu.VMEM((CHUNK, DIM), jnp.float32),
            pltpu.VMEM((CHUNK, DIM), jnp.float32),
            pltpu.SemaphoreType.DMA,
            pltpu.SemaphoreType.DMA,
        ],
        compiler_params=pltpu.CompilerParams(use_tc_tiling_on_sc=False),
    )
    def k(table_hbm, idx_hbm, out_hbm, idx_v, rows0, rows1, gsem, wsem):
        c = lax.axis_index("c")
        s = lax.axis_index("s")
        wid = s * 2 + c
        base = wid * per_w
        rows = [rows0, rows1]

        pltpu.sync_copy(idx_hbm.at[pl.ds(wid * n_chunks, n_chunks)], idx_v)

        gathers = [None] * n_chunks
        wbs = [None] * n_chunks
        gathers[0] = pltpu.async_copy(table_hbm.at[idx_v.at[0]], rows[0], gsem)
        for i in range(n_chunks):
            gathers[i].wait()
            wbs[i] = pltpu.async_copy(
                rows[i % 2], out_hbm.at[pl.ds(base + i * CHUNK, CHUNK)], wsem
            )
            if i + 1 < n_chunks:
                if i - 1 >= 0:
                    wbs[i - 1].wait()
                gathers[i + 1] = pltpu.async_copy(
                    table_hbm.at[idx_v.at[i + 1]], rows[(i + 1) % 2], gsem
                )
        if n_chunks >= 2:
            wbs[n_chunks - 2].wait()
        wbs[n_chunks - 1].wait()

    return k


def kernel(input_, table):
    B, L = input_.shape
    n_total = B * L
    n_chunks = (n_total // NUM_WORKERS) // CHUNK
    idx = input_.reshape(NUM_WORKERS * n_chunks, CHUNK).astype(jnp.int32)
    out = _emb_kernel(n_total)(table, idx)
    return out.reshape(B, L, DIM)


# speedup vs baseline: 1.4634x; 1.4634x over previous
import functools

import jax
import jax.numpy as jnp
import numpy as np
from jax import lax
from jax.experimental import pallas as pl
from jax.experimental.pallas import tpu as pltpu
from jax.experimental.pallas import tpu_sc as plsc

DIM = 32
NUM_WORKERS = 32
BT = 128
L = 50
NBT = 128
BT_PER_W = NBT // NUM_WORKERS
BLOCKS_PER_W = BT_PER_W * L

GATHER_BYTES = BT * 128 * 4
WB_BYTES = 4 * 8 * BT * 4


def _emb_kernel():
    mesh = plsc.VectorSubcoreMesh(core_axis_name="c", subcore_axis_name="s")

    @functools.partial(
        pl.kernel,
        mesh=mesh,
        out_type=jax.ShapeDtypeStruct((L, 4, NBT, 8, BT), jnp.float32),
        scratch_types=[
            pltpu.VMEM((BT_PER_W, L, BT), jnp.int32),
            pltpu.VMEM((BT_PER_W, L, BT), jnp.int32),
            pltpu.VMEM((BT, 128), jnp.float32),
            pltpu.VMEM((BT, 128), jnp.float32),
            pltpu.VMEM((DIM, BT), jnp.float32),
            pltpu.VMEM((DIM, BT), jnp.float32),
            pltpu.SemaphoreType.DMA,
            pltpu.SemaphoreType.DMA,
        ],
        compiler_params=pltpu.CompilerParams(
            use_tc_tiling_on_sc=False, needs_layout_passes=False
        ),
    )
    def k(tableq, idxq, out5, idx_v, idxs_v, rows0, rows1, tr0, tr1, gsem, wsem):
        c = lax.axis_index("c")
        s = lax.axis_index("s")
        wid = s * 2 + c
        rows = [rows0, rows1]
        trs = [tr0, tr1]

        pltpu.sync_copy(idxq.at[pl.ds(wid * BT_PER_W, BT_PER_W)], idx_v)

        def shift_chunk(i, carry):
            btk = i // (L * 8)
            r = lax.rem(i, L * 8)
            li = r // 8
            ch = lax.rem(r, 8)
            v = idx_v[btk, li, pl.ds(ch * 16, 16)]
            idxs_v[btk, li, pl.ds(ch * 16, 16)] = lax.shift_right_logical(v, 2)
            return carry

        lax.fori_loop(0, BT_PER_W * L * 8, shift_chunk, 0)

        iota16 = lax.iota(jnp.int32, 16)

        def issue_gather(b, buf):
            btk = b // L
            li = lax.rem(b, L)
            return pltpu.async_copy(
                tableq.at[idxs_v.at[btk, li]], rows[buf], gsem
            )

        def do_block(si, par):
            b = si * 2 + par
            btk = b // L
            li = lax.rem(b, L)
            pltpu.make_async_copy(
                tableq.at[idxs_v.at[btk, li]], rows[par], gsem
            ).wait()
            if par == 0:
                issue_gather(b + 1, 1 - par)
            else:
                @pl.when(si < BLOCKS_PER_W // 2 - 1)
                def _():
                    issue_gather(b + 1, 1 - par)

            rowsb = rows[par]
            trb = trs[par]

            @pl.when(si >= 1)
            def _():
                for dt in range(4):
                    pltpu.make_async_copy(
                        trb.at[pl.ds(dt * 8, 8)], out5.at[li, dt, 0], wsem
                    ).wait()

            cols = []
            rvecs = []
            for ch in range(8):
                iv = idx_v[btk, li, pl.ds(ch * 16, 16)]
                cols.append(lax.shift_left(lax.bitwise_and(iv, 3), 5))
                rvecs.append(iota16 + (ch * 16))

            def tbody(cc, carry):
                for ch in range(8):
                    v = plsc.load_gather(rowsb, [rvecs[ch], cols[ch] + cc])
                    trb[cc, pl.ds(ch * 16, 16)] = v
                return carry

            lax.fori_loop(0, DIM, tbody, 0)

            bt_abs = wid * BT_PER_W + btk
            for dt in range(4):
                pltpu.async_copy(
                    trb.at[pl.ds(dt * 8, 8)], out5.at[li, dt, bt_abs], wsem
                )

        issue_gather(0, 0)

        def loop_body(si, carry):
            do_block(si, 0)
            do_block(si, 1)
            return carry

        lax.fori_loop(0, BLOCKS_PER_W // 2, loop_body, 0)

        for par in range(2):
            for dt in range(4):
                pltpu.make_async_copy(
                    trs[par].at[pl.ds(dt * 8, 8)], out5.at[0, dt, 0], wsem
                ).wait()

    return k


def kernel(input_, table):
    B, _ = input_.shape
    tableq = table.reshape(250000, 128)
    idxq = (
        input_.astype(jnp.int32)
        .T.reshape(L, NBT, BT)
        .transpose(1, 0, 2)
    )
    out5 = _emb_kernel()(tableq, idxq)
    return out5.transpose(2, 4, 0, 1, 3).reshape(B, L, DIM)


# speedup vs baseline: 1.4840x; 1.0141x over previous
import functools

import jax
import jax.numpy as jnp
from jax import lax
from jax.experimental import pallas as pl
from jax.experimental.pallas import tpu as pltpu
from jax.experimental.pallas import tpu_sc as plsc

DIM = 32
NUM_WORKERS = 32
BT = 128
L = 50
NBT = 128
BT_PER_W = NBT // NUM_WORKERS
BLOCKS_PER_W = BT_PER_W * L


def _emb_kernel():
    mesh = plsc.VectorSubcoreMesh(core_axis_name="c", subcore_axis_name="s")

    @functools.partial(
        pl.kernel,
        mesh=mesh,
        out_type=jax.ShapeDtypeStruct((L, 4, NBT, 8, BT), jnp.float32),
        scratch_types=[
            pltpu.VMEM((BT_PER_W, L, BT), jnp.int32),
            pltpu.VMEM((BT, DIM), jnp.float32),
            pltpu.VMEM((BT, DIM), jnp.float32),
            pltpu.VMEM((DIM, BT), jnp.float32),
            pltpu.VMEM((DIM, BT), jnp.float32),
            pltpu.SemaphoreType.DMA,
            pltpu.SemaphoreType.DMA,
        ],
        compiler_params=pltpu.CompilerParams(
            use_tc_tiling_on_sc=False, needs_layout_passes=False
        ),
    )
    def k(table, idxq, out5, idx_v, rows0, rows1, tr0, tr1, gsem, wsem):
        c = lax.axis_index("c")
        s = lax.axis_index("s")
        wid = s * 2 + c
        rows = [rows0, rows1]
        trs = [tr0, tr1]

        pltpu.sync_copy(idxq.at[pl.ds(wid * BT_PER_W, BT_PER_W)], idx_v)

        iota16 = lax.iota(jnp.int32, 16)
        rvecs = [iota16 + (ch * 16) for ch in range(8)]

        def issue_gather(b, buf):
            btk = b // L
            li = lax.rem(b, L)
            return pltpu.async_copy(table.at[idx_v.at[btk, li]], rows[buf], gsem)

        def do_block(si, par):
            b = si * 2 + par
            btk = b // L
            li = lax.rem(b, L)
            pltpu.make_async_copy(
                table.at[idx_v.at[btk, li]], rows[par], gsem
            ).wait()
            if par == 0:
                issue_gather(b + 1, 1 - par)
            else:
                @pl.when(si < BLOCKS_PER_W // 2 - 1)
                def _():
                    issue_gather(b + 1, 1 - par)

            rowsb = rows[par]
            trb = trs[par]

            @pl.when(si >= 1)
            def _():
                for dt in range(4):
                    pltpu.make_async_copy(
                        trb.at[pl.ds(dt * 8, 8)], out5.at[li, dt, 0], wsem
                    ).wait()

            def tbody(cc, carry):
                col = jnp.broadcast_to(cc, (16,)).astype(jnp.int32)
                for ch in range(8):
                    v = plsc.load_gather(rowsb, [rvecs[ch], col])
                    trb[cc, pl.ds(ch * 16, 16)] = v
                return carry

            lax.fori_loop(0, DIM, tbody, 0)

            bt_abs = wid * BT_PER_W + btk
            for dt in range(4):
                pltpu.async_copy(
                    trb.at[pl.ds(dt * 8, 8)], out5.at[li, dt, bt_abs], wsem
                )

        issue_gather(0, 0)

        def loop_body(si, carry):
            do_block(si, 0)
            do_block(si, 1)
            return carry

        lax.fori_loop(0, BLOCKS_PER_W // 2, loop_body, 0)

        for par in range(2):
            for dt in range(4):
                pltpu.make_async_copy(
                    trs[par].at[pl.ds(dt * 8, 8)], out5.at[0, dt, 0], wsem
                ).wait()

    return k


def kernel(input_, table):
    B, _ = input_.shape
    idxq = (
        input_.astype(jnp.int32)
        .T.reshape(L, NBT, BT)
        .transpose(1, 0, 2)
    )
    out5 = _emb_kernel()(table, idxq)
    return out5.transpose(2, 4, 0, 1, 3).reshape(B, L, DIM)
